# Initial kernel scaffold; baseline (speedup 1.0000x reference)
import jax
import jax.numpy as jnp
from jax import lax
from jax.experimental import pallas as pl
from jax.experimental.pallas import tpu as pltpu
from jax.experimental.pallas import tpu_sc as plsc

B = 16384
F = 128
NC = 2
NS = 16
L = 16
NW = NC * NS
BPW = B // NW
CH = 128
NCHUNK = BPW // CH


def _body(user_hbm, item_hbm, uf_hbm, if_hbm, out_hbm,
          uidx_v, iidx_v, ubuf, vbuf, outv, sem_u, sem_v):
    c = lax.axis_index("c")
    s = lax.axis_index("s")
    wid = s * NC + c
    base = wid * BPW

    pltpu.sync_copy(user_hbm.at[pl.ds(base, BPW)], uidx_v)
    pltpu.sync_copy(item_hbm.at[pl.ds(base, BPW)], iidx_v)

    lane = lax.iota(jnp.int32, L)
    for ch in range(NCHUNK):
        cu = pltpu.async_copy(uf_hbm.at[uidx_v.at[pl.ds(ch * CH, CH)]], ubuf, sem_u)
        cv = pltpu.async_copy(if_hbm.at[iidx_v.at[pl.ds(ch * CH, CH)]], vbuf, sem_v)
        cu.wait()
        cv.wait()
        for g in range(CH // L):
            rows = lane + g * L

            def f_body(_, carry, rows=rows):
                acc, fv = carry
                for _k in range(8):
                    ug = plsc.load_gather(ubuf, [rows, fv])
                    vg = plsc.load_gather(vbuf, [rows, fv])
                    acc = acc + ug * vg
                    fv = fv + 1
                return acc, fv

            acc, _ = lax.fori_loop(
                0, F // 8, f_body,
                (jnp.zeros((L,), jnp.float32), jnp.zeros((L,), jnp.int32)))
            outv[pl.ds(ch * CH + g * L, L)] = acc

    pltpu.sync_copy(outv, out_hbm.at[pl.ds(base, BPW)])


def kernel(user, item, user_factors, item_factors):
    mesh = plsc.VectorSubcoreMesh(core_axis_name="c", subcore_axis_name="s")
    k = pl.kernel(
        _body,
        out_type=jax.ShapeDtypeStruct((B,), jnp.float32),
        mesh=mesh,
        scratch_types=[
            pltpu.VMEM((BPW,), jnp.int32),
            pltpu.VMEM((BPW,), jnp.int32),
            pltpu.VMEM((CH, F), jnp.float32),
            pltpu.VMEM((CH, F), jnp.float32),
            pltpu.VMEM((BPW,), jnp.float32),
            pltpu.SemaphoreType.DMA,
            pltpu.SemaphoreType.DMA,
        ],
    )
    return k(user, item, user_factors, item_factors)


# speedup vs baseline: 1.0293x; 1.0293x over previous
import jax
import jax.numpy as jnp
from jax import lax
from jax.experimental import pallas as pl
from jax.experimental.pallas import tpu as pltpu
from jax.experimental.pallas import tpu_sc as plsc

B = 16384
F = 128
NC = 2
NS = 16
L = 16
NW = NC * NS
BPW = B // NW
CH = 128
NCHUNK = BPW // CH


def _body(user_hbm, item_hbm, uf_hbm, if_hbm, out_hbm,
          uidx_v, iidx_v, ubuf, vbuf, outv, stage, sem_u, sem_v):
    c = lax.axis_index("c")
    s = lax.axis_index("s")
    wid = s * NC + c
    base = wid * BPW

    pltpu.sync_copy(user_hbm.at[pl.ds(base, BPW)], uidx_v)
    pltpu.sync_copy(item_hbm.at[pl.ds(base, BPW)], iidx_v)

    lane = lax.iota(jnp.int32, L)
    shuffle_idx = [(lane ^ sh)[:, None] for sh in (8, 4, 2, 1)]
    dn = lax.GatherDimensionNumbers(
        offset_dims=(), collapsed_slice_dims=(0,), start_index_map=(0,))

    for ch in range(NCHUNK):
        cu = pltpu.async_copy(uf_hbm.at[uidx_v.at[pl.ds(ch * CH, CH)]], ubuf, sem_u)
        cv = pltpu.async_copy(if_hbm.at[iidx_v.at[pl.ds(ch * CH, CH)]], vbuf, sem_v)
        cu.wait()
        cv.wait()

        def g_body(g, _, ch=ch):
            out16 = jnp.zeros((L,), jnp.float32)
            for j in range(L):
                r = g * L + j
                acc = ubuf[r, pl.ds(0, L)] * vbuf[r, pl.ds(0, L)]
                for k in range(1, F // L):
                    acc = acc + ubuf[r, pl.ds(k * L, L)] * vbuf[r, pl.ds(k * L, L)]
                for sidx in shuffle_idx:
                    acc = acc + lax.gather(
                        acc, sidx, dn, (1,),
                        mode=lax.GatherScatterMode.PROMISE_IN_BOUNDS)
                out16 = jnp.where(lane == j, acc, out16)
            outv[pl.ds(ch * CH + g * L, L)] = out16
            return 0

        lax.fori_loop(0, CH // L, g_body, 0)

    pltpu.sync_copy(outv, out_hbm.at[pl.ds(base, BPW)])


def kernel(user, item, user_factors, item_factors):
    mesh = plsc.VectorSubcoreMesh(core_axis_name="c", subcore_axis_name="s")
    k = pl.kernel(
        _body,
        out_type=jax.ShapeDtypeStruct((B,), jnp.float32),
        mesh=mesh,
        scratch_types=[
            pltpu.VMEM((BPW,), jnp.int32),
            pltpu.VMEM((BPW,), jnp.int32),
            pltpu.VMEM((CH, F), jnp.float32),
            pltpu.VMEM((CH, F), jnp.float32),
            pltpu.VMEM((BPW,), jnp.float32),
            pltpu.VMEM((L * L,), jnp.float32),
            pltpu.SemaphoreType.DMA,
            pltpu.SemaphoreType.DMA,
        ],
    )
    return k(user, item, user_factors, item_factors)
